# Initial kernel scaffold; baseline (speedup 1.0000x reference)
import jax
import jax.numpy as jnp
from jax.experimental import pallas as pl


def kernel(x, graph_index, gamma1, beta1, W1, b1, W2, b2, gamma2, beta2):
    raise NotImplementedError("write your pallas kernel here")


# speedup vs baseline: 7.6031x; 7.6031x over previous
import functools

import jax
import jax.numpy as jnp
from jax import lax
from jax.experimental import pallas as pl
from jax.experimental.pallas import tpu as pltpu
from jax.experimental.pallas import tpu_sc as plsc

N = 320000
D = 128
S = 1024

NC = 2
NS = 16
NW = NC * NS

B = 512
SUB = 128
NSUB = B // SUB
NB = N // B
KMAX = -(-NB // NW)
ROWS_PER_TILE = S // NS

_mesh = plsc.VectorSubcoreMesh(core_axis_name="c", subcore_axis_name="s")


@functools.partial(
    pl.kernel,
    mesh=_mesh,
    out_type=[
        jax.ShapeDtypeStruct((NC, S, D), jnp.float32),
        jax.ShapeDtypeStruct((NC, S, 16), jnp.float32),
    ],
    scratch_types=[
        pltpu.VMEM((B, D), jnp.float32),
        pltpu.VMEM((NSUB, SUB), jnp.int32),
        pltpu.VMEM((SUB, 16), jnp.float32),
        pltpu.VMEM_SHARED((S, D), jnp.float32),
        pltpu.VMEM_SHARED((S, 16), jnp.float32),
    ],
)
def _sc_segment_sum(x_hbm, idx_hbm, zs_hbm, zc_hbm, ones_hbm,
                    sums_out, counts_out,
                    rowbuf, idxbuf, onesbuf, acc, accc):
    c = lax.axis_index("c")
    s = lax.axis_index("s")
    wid = s * NC + c

    r0 = s * ROWS_PER_TILE
    pltpu.sync_copy(zs_hbm.at[pl.ds(r0, ROWS_PER_TILE)], acc.at[pl.ds(r0, ROWS_PER_TILE)])
    pltpu.sync_copy(zc_hbm.at[pl.ds(r0, ROWS_PER_TILE)], accc.at[pl.ds(r0, ROWS_PER_TILE)])
    pltpu.sync_copy(ones_hbm, onesbuf)
    plsc.subcore_barrier()

    def block_body(k, carry):
        b = k * NW + wid

        @pl.when(b < NB)
        def _():
            pltpu.sync_copy(x_hbm.at[pl.ds(b * B, B)], rowbuf)
            pltpu.sync_copy(idx_hbm.at[pl.ds(b * NSUB, NSUB)], idxbuf)
            for j in range(NSUB):
                pltpu.sync_copy(rowbuf.at[pl.ds(j * SUB, SUB)],
                                acc.at[idxbuf.at[j]], add=True)
                pltpu.sync_copy(onesbuf, accc.at[idxbuf.at[j]], add=True)

        return carry

    lax.fori_loop(0, KMAX, block_body, 0)

    plsc.subcore_barrier()
    pltpu.sync_copy(acc.at[pl.ds(r0, ROWS_PER_TILE)],
                    sums_out.at[c, pl.ds(r0, ROWS_PER_TILE)])
    pltpu.sync_copy(accc.at[pl.ds(r0, ROWS_PER_TILE)],
                    counts_out.at[c, pl.ds(r0, ROWS_PER_TILE)])


def _tc_head(sums_ref, counts_ref, g1_ref, be1_ref, w1_ref, b1_ref,
             w2_ref, b2_ref, g2_ref, be2_ref, out_ref):
    sums = sums_ref[0, :, :] + sums_ref[1, :, :]
    counts = counts_ref[0, :, :] + counts_ref[1, :, :]
    cnt = jnp.maximum(counts[:, 0:1], 1.0)
    h = sums / cnt

    def layer_norm(v, gamma, beta):
        mean = jnp.mean(v, axis=-1, keepdims=True)
        var = jnp.var(v, axis=-1, keepdims=True)
        return (v - mean) * lax.rsqrt(var + 1e-5) * gamma + beta

    h = layer_norm(h, g1_ref[0:1, :], be1_ref[0:1, :])
    y = lax.dot_general(h, w1_ref[:, :], (((1,), (1,)), ((), ())),
                        preferred_element_type=jnp.float32,
                        precision=lax.Precision.HIGHEST) + b1_ref[0:1, :]
    y = jnp.where(y > 0, y, jnp.exp(jnp.minimum(y, 0.0)) - 1.0)
    y = lax.dot_general(y, w2_ref[:, :], (((1,), (1,)), ((), ())),
                        preferred_element_type=jnp.float32,
                        precision=lax.Precision.HIGHEST) + b2_ref[0:1, :]
    y = y + h
    out_ref[:, :] = layer_norm(y, g2_ref[0:1, :], be2_ref[0:1, :])


_tc_head_call = pl.pallas_call(
    _tc_head,
    out_shape=jax.ShapeDtypeStruct((S, D), jnp.float32),
)


@jax.jit
def kernel(x, graph_index, gamma1, beta1, W1, b1, W2, b2, gamma2, beta2):
    idx = graph_index.astype(jnp.int32).reshape(N // SUB, SUB)
    zeros_s = jnp.zeros((S, D), jnp.float32)
    zeros_c = jnp.zeros((S, 16), jnp.float32)
    ones_b = jnp.ones((SUB, 16), jnp.float32)
    sums, counts = _sc_segment_sum(x, idx, zeros_s, zeros_c, ones_b)
    return _tc_head_call(
        sums, counts,
        gamma1.reshape(1, D), beta1.reshape(1, D), W1, b1.reshape(1, D),
        W2, b2.reshape(1, D), gamma2.reshape(1, D), beta2.reshape(1, D))


# speedup vs baseline: 7.6264x; 1.0031x over previous
import functools

import jax
import jax.numpy as jnp
from jax import lax
from jax.experimental import pallas as pl
from jax.experimental.pallas import tpu as pltpu
from jax.experimental.pallas import tpu_sc as plsc

N = 320000
D = 128
S = 1024

NC = 2
NS = 16
NW = NC * NS

ROWS_PER_WORKER = N // NW
B = 400
NBW = ROWS_PER_WORKER // B
SUB = 80
NSUB = B // SUB
ROWS_PER_TILE = S // NS

_mesh = plsc.VectorSubcoreMesh(core_axis_name="c", subcore_axis_name="s")


@functools.partial(
    pl.kernel,
    mesh=_mesh,
    out_type=[
        jax.ShapeDtypeStruct((NC, S, D), jnp.float32),
        jax.ShapeDtypeStruct((NC, S, 16), jnp.float32),
    ],
    scratch_types=[
        pltpu.VMEM((B, D), jnp.float32),
        pltpu.VMEM((B, D), jnp.float32),
        pltpu.VMEM((NSUB, SUB), jnp.int32),
        pltpu.VMEM((NSUB, SUB), jnp.int32),
        pltpu.VMEM((SUB, 16), jnp.float32),
        pltpu.VMEM_SHARED((S, D), jnp.float32),
        pltpu.VMEM_SHARED((S, 16), jnp.float32),
        pltpu.SemaphoreType.DMA,
        pltpu.SemaphoreType.DMA,
        pltpu.SemaphoreType.DMA,
    ],
)
def _sc_segment_sum(x_hbm, idx_hbm, zs_hbm, zc_hbm, ones_hbm,
                    sums_out, counts_out,
                    rowbuf0, rowbuf1, idxbuf0, idxbuf1, onesbuf, acc, accc,
                    gsem0, gsem1, ssem):
    c = lax.axis_index("c")
    s = lax.axis_index("s")
    wid = s * NC + c
    blk0 = wid * NBW

    rowbufs = (rowbuf0, rowbuf1)
    idxbufs = (idxbuf0, idxbuf1)
    gsems = (gsem0, gsem1)

    r0 = s * ROWS_PER_TILE
    pltpu.sync_copy(zs_hbm.at[pl.ds(r0, ROWS_PER_TILE)], acc.at[pl.ds(r0, ROWS_PER_TILE)])
    pltpu.sync_copy(zc_hbm.at[pl.ds(r0, ROWS_PER_TILE)], accc.at[pl.ds(r0, ROWS_PER_TILE)])
    pltpu.sync_copy(ones_hbm, onesbuf)
    plsc.subcore_barrier()

    def row_base(j):
        return pl.multiple_of((blk0 + j) * B, 16)

    def gather_start(j, p):
        pltpu.async_copy(x_hbm.at[pl.ds(row_base(j), B)], rowbufs[p], gsems[p])
        pltpu.async_copy(idx_hbm.at[blk0 + j], idxbufs[p], gsems[p])

    def gather_wait(j, p):
        pltpu.make_async_copy(x_hbm.at[pl.ds(row_base(j), B)], rowbufs[p],
                              gsems[p]).wait()
        pltpu.make_async_copy(idx_hbm.at[blk0 + j], idxbufs[p],
                              gsems[p]).wait()

    def scatter_block(j, p):
        buf = rowbufs[p]
        idxb = idxbufs[p]
        cps = []
        for t in range(NSUB):
            cps.append(pltpu.async_copy(
                buf.at[pl.ds(t * SUB, SUB)],
                acc.at[idxb.at[t]], ssem, add=True))
            cps.append(pltpu.async_copy(
                onesbuf, accc.at[idxb.at[t]], ssem, add=True))
        for cp in cps:
            cp.wait()

    gather_start(0, 0)

    def pair_body(k2, carry):
        j0 = k2 * 2
        for p in range(2):
            j = j0 + p
            gather_wait(j, p)
            gather_start(j + 1, 1 - p)
            scatter_block(j, p)
        return carry

    lax.fori_loop(0, (NBW - 1) // 2, pair_body, 0)
    gather_wait(NBW - 1, 0)
    scatter_block(NBW - 1, 0)

    plsc.subcore_barrier()
    pltpu.sync_copy(acc.at[pl.ds(r0, ROWS_PER_TILE)],
                    sums_out.at[c, pl.ds(r0, ROWS_PER_TILE)])
    pltpu.sync_copy(accc.at[pl.ds(r0, ROWS_PER_TILE)],
                    counts_out.at[c, pl.ds(r0, ROWS_PER_TILE)])


def _tc_head(sums_ref, counts_ref, g1_ref, be1_ref, w1_ref, b1_ref,
             w2_ref, b2_ref, g2_ref, be2_ref, out_ref):
    sums = sums_ref[0, :, :] + sums_ref[1, :, :]
    counts = counts_ref[0, :, :] + counts_ref[1, :, :]
    cnt = jnp.maximum(counts[:, 0:1], 1.0)
    h = sums / cnt

    def layer_norm(v, gamma, beta):
        mean = jnp.mean(v, axis=-1, keepdims=True)
        var = jnp.var(v, axis=-1, keepdims=True)
        return (v - mean) * lax.rsqrt(var + 1e-5) * gamma + beta

    h = layer_norm(h, g1_ref[0:1, :], be1_ref[0:1, :])
    y = lax.dot_general(h, w1_ref[:, :], (((1,), (1,)), ((), ())),
                        preferred_element_type=jnp.float32,
                        precision=lax.Precision.HIGHEST) + b1_ref[0:1, :]
    y = jnp.where(y > 0, y, jnp.exp(jnp.minimum(y, 0.0)) - 1.0)
    y = lax.dot_general(y, w2_ref[:, :], (((1,), (1,)), ((), ())),
                        preferred_element_type=jnp.float32,
                        precision=lax.Precision.HIGHEST) + b2_ref[0:1, :]
    y = y + h
    out_ref[:, :] = layer_norm(y, g2_ref[0:1, :], be2_ref[0:1, :])


_tc_head_call = pl.pallas_call(
    _tc_head,
    out_shape=jax.ShapeDtypeStruct((S, D), jnp.float32),
)


@jax.jit
def kernel(x, graph_index, gamma1, beta1, W1, b1, W2, b2, gamma2, beta2):
    idx = graph_index.astype(jnp.int32).reshape(N // B, NSUB, SUB)
    zeros_s = jnp.zeros((S, D), jnp.float32)
    zeros_c = jnp.zeros((S, 16), jnp.float32)
    ones_b = jnp.ones((SUB, 16), jnp.float32)
    sums, counts = _sc_segment_sum(x, idx, zeros_s, zeros_c, ones_b)
    return _tc_head_call(
        sums, counts,
        gamma1.reshape(1, D), beta1.reshape(1, D), W1, b1.reshape(1, D),
        W2, b2.reshape(1, D), gamma2.reshape(1, D), beta2.reshape(1, D))
